# baseline (speedup 1.0000x reference)
import jax
import jax.numpy as jnp
from jax import lax
from jax.experimental import pallas as pl
from jax.experimental.pallas import tpu as pltpu
from jax.experimental.pallas import tpu_sc as plsc

_N_USERS = 5000
_N_NODES = 10000
_D = 128
_E = 320000
_EPS = 1e-5

_NC = 2
_NS = 16
_NW = _NC * _NS
_K = 80
_NB = 126
_EPW = _K * _NB
_EP = _NW * _EPW
_NPAD = 10240
_RPT = _NPAD // _NS
_CH = 18
_NCHK = _NB // _CH
_CHE = _CH * _K


def _sc_hop_body(emb_hbm, rows_hbm, cols_hbm, vals_hbm, out_hbm,
                 acc, rows_ch, cols_ch, vals_ch, gbf, g32, sem_g, sem_s):
    c = lax.axis_index("c")
    s = lax.axis_index("s")
    w = s * _NC + c

    z16 = jnp.zeros((16,), jnp.float32)

    @pl.loop(0, _K)
    def _zero_rows(r):
        for cc in range(_D // 16):
            g32[0, r, pl.ds(cc * 16, 16)] = z16

    for k in range(_RPT // _K):
        pltpu.sync_copy(g32.at[0], acc.at[pl.ds(s * _RPT + k * _K, _K)])
    plsc.subcore_barrier()

    def _issue_gather(b, p):
        return pltpu.async_copy(emb_hbm.at[cols_ch.at[pl.ds(b * _K, _K)]],
                                gbf.at[p], sem_g.at[p])

    def _wait_gather(b, p):
        pltpu.make_async_copy(emb_hbm.at[cols_ch.at[pl.ds(b * _K, _K)]],
                              gbf.at[p], sem_g.at[p]).wait()

    def _drain_scatter(p):
        pltpu.make_async_copy(g32.at[p], acc.at[rows_ch.at[0]],
                              sem_s.at[p]).wait()

    def _scale(b, p):
        @pl.loop(0, _K // 16)
        def _grp(g):
            v16 = vals_ch[pl.ds(b * _K + g * 16, 16)]
            for j in range(16):
                idx = jnp.full((16,), j, jnp.int32)
                vb = v16.at[idx].get(mode="promise_in_bounds")
                row = g * 16 + j
                for cc in range(_D // 32):
                    raw = gbf[p, row, pl.ds(cc * 16, 16)]
                    lo = plsc.bitcast(raw << 16, jnp.float32)
                    hi = plsc.bitcast(raw & jnp.int32(-65536), jnp.float32)
                    g32[p, row, pl.ds(cc * 32, 16)] = lo * vb
                    g32[p, row, pl.ds(cc * 32 + 16, 16)] = hi * vb

    def _step(b, p):
        _wait_gather(b, p)

        @pl.when(b >= 2)
        def _drain_prev():
            _drain_scatter(p)

        _scale(b, p)
        pltpu.async_copy(g32.at[p], acc.at[rows_ch.at[b]], sem_s.at[p],
                         add=True)

        @pl.when(b + 2 < _CH)
        def _prefetch():
            _issue_gather(b + 2, p)

    for ch in range(_NCHK):
        pltpu.sync_copy(rows_hbm.at[w * _NCHK + ch], rows_ch)
        pltpu.sync_copy(cols_hbm.at[pl.ds(w * _EPW + ch * _CHE, _CHE)],
                        cols_ch)
        pltpu.sync_copy(vals_hbm.at[pl.ds(w * _EPW + ch * _CHE, _CHE)],
                        vals_ch)

        _issue_gather(0, 0)
        _issue_gather(1, 1)

        @pl.loop(0, _CH // 2)
        def _main(i):
            b0 = i * 2
            for u in range(2):
                _step(b0 + u, u)

        _drain_scatter((_CH - 2) % 2)
        _drain_scatter((_CH - 1) % 2)

    plsc.subcore_barrier()

    r0 = s * _RPT
    pltpu.sync_copy(acc.at[pl.ds(r0, _RPT)], out_hbm.at[c, pl.ds(r0, _RPT)])


def _sc_hop(embp, rows3, cols1, vals1):
    return pl.kernel(
        _sc_hop_body,
        out_type=jax.ShapeDtypeStruct((_NC, _NPAD, _D), jnp.float32),
        mesh=plsc.VectorSubcoreMesh(core_axis_name="c", subcore_axis_name="s"),
        compiler_params=pltpu.CompilerParams(needs_layout_passes=False,
                                             use_tc_tiling_on_sc=False),
        scratch_types=[
            pltpu.VMEM_SHARED((_NPAD, _D), jnp.float32),
            pltpu.VMEM((_CH, _K), jnp.int32),
            pltpu.VMEM((_CHE,), jnp.int32),
            pltpu.VMEM((_CHE,), jnp.float32),
            pltpu.VMEM((2, _K, _D // 2), jnp.int32),
            pltpu.VMEM((2, _K, _D), jnp.float32),
            pltpu.SemaphoreType.DMA((2,)),
            pltpu.SemaphoreType.DMA((2,)),
        ],
    )(embp, rows3, cols1, vals1)


def _pack_table(y):
    bm = y.shape[0]
    y4 = y.reshape(bm, _D // 32, 2, 16)

    def _bf_bits(v):
        return lax.bitcast_convert_type(
            v.astype(jnp.bfloat16).astype(jnp.float32), jnp.int32)

    a = _bf_bits(y4[:, :, 0, :])
    b = _bf_bits(y4[:, :, 1, :])
    packed = lax.shift_right_logical(a, 16) | (b & jnp.int32(-65536))
    return packed.reshape(bm, _D // 2)


def _ln_body(p_ref, g_ref, b_ref, o_ref, opk_ref):
    x = p_ref[0] + p_ref[1]
    mu = jnp.mean(x, axis=-1, keepdims=True)
    xc = x - mu
    var = jnp.mean(xc * xc, axis=-1, keepdims=True)
    y = xc * lax.rsqrt(var + _EPS) * g_ref[...] + b_ref[...]
    o_ref[...] = y
    opk_ref[...] = _pack_table(y)


def _tc_ln(partials, gamma, beta):
    bm = 1000
    return pl.pallas_call(
        _ln_body,
        out_shape=[jax.ShapeDtypeStruct((_N_NODES, _D), jnp.float32),
                   jax.ShapeDtypeStruct((_N_NODES, _D // 2), jnp.int32)],
        grid=(_N_NODES // bm,),
        in_specs=[
            pl.BlockSpec((_NC, bm, _D), lambda i: (0, i, 0)),
            pl.BlockSpec((1, _D), lambda i: (0, 0)),
            pl.BlockSpec((1, _D), lambda i: (0, 0)),
        ],
        out_specs=[pl.BlockSpec((bm, _D), lambda i: (i, 0)),
                   pl.BlockSpec((bm, _D // 2), lambda i: (i, 0))],
    )(partials, gamma.reshape(1, _D), beta.reshape(1, _D))


def kernel(user_embed, item_embed, edge_index, edge_values, gamma, beta):
    all_embed = jnp.concatenate([user_embed, item_embed], axis=0)
    npad_e = _EP - _E
    pad_rows = _N_NODES + (jnp.arange(npad_e, dtype=jnp.int32)
                           % (_NPAD - _N_NODES))
    rows = jnp.concatenate(
        [edge_index[0].astype(jnp.int32), pad_rows]).reshape(
            _NW * _NCHK, _CH, _K)
    cols = jnp.concatenate(
        [edge_index[1].astype(jnp.int32),
         jnp.arange(npad_e, dtype=jnp.int32) % _N_NODES])
    vals = jnp.concatenate([edge_values, jnp.zeros((npad_e,), jnp.float32)])

    packed = _pack_table(all_embed)
    embs = [all_embed]
    for _ in range(3):
        partials = _sc_hop(packed, rows, cols, vals)
        agg, packed = _tc_ln(partials[:, :_N_NODES], gamma, beta)
        embs.append(agg)
    embs = jnp.stack(embs, axis=1)
    return embs[:_N_USERS], embs[_N_USERS:]


# speedup vs baseline: 2.1867x; 2.1867x over previous
import jax
import jax.numpy as jnp
from jax import lax
from jax.experimental import pallas as pl
from jax.experimental.pallas import tpu as pltpu
from jax.experimental.pallas import tpu_sc as plsc

_N_USERS = 5000
_N_NODES = 10000
_D = 128
_E = 320000
_EPS = 1e-5

_NC = 2
_NS = 16
_NW = _NC * _NS
_K = 96
_NB = 105
_EPW = _K * _NB
_EP = _NW * _EPW
_NPAD = 10240
_RPT = _NPAD // _NS
_CH = 35
_NCHK = _NB // _CH
_CHE = _CH * _K


def _sc_hop_body(emb_hbm, rows_hbm, cols_hbm, vals_hbm, out_hbm,
                 acc, rows_ch, cols_ch, vals_ch, gath, sem_g, sem_s, sem_i):
    c = lax.axis_index("c")
    s = lax.axis_index("s")
    w = s * _NC + c

    cp_r = pltpu.async_copy(rows_hbm.at[w * _NCHK], rows_ch, sem_i)
    cp_c = pltpu.async_copy(cols_hbm.at[pl.ds(w * _EPW, _CHE)], cols_ch,
                            sem_i)
    cp_v = pltpu.async_copy(vals_hbm.at[pl.ds(w * _EPW, _CHE)], vals_ch,
                            sem_i)

    z16 = jnp.zeros((16,), jnp.float32)

    @pl.loop(0, _K)
    def _zero_rows(r):
        for cc in range(_D // 16):
            gath[0, r, pl.ds(cc * 16, 16)] = z16

    for k in range(_RPT // _K):
        pltpu.sync_copy(gath.at[0], acc.at[pl.ds(s * _RPT + k * _K, _K)])
    _rem = _RPT - (_RPT // _K) * _K
    if _rem:
        pltpu.sync_copy(gath.at[0, pl.ds(0, _rem)],
                        acc.at[pl.ds(s * _RPT + (_RPT // _K) * _K, _rem)])
    cp_r.wait()
    cp_c.wait()
    cp_v.wait()
    plsc.subcore_barrier()

    def _issue_gather(b, p):
        return pltpu.async_copy(emb_hbm.at[cols_ch.at[pl.ds(b * _K, _K)]],
                                gath.at[p], sem_g.at[p])

    def _wait_gather(b, p):
        pltpu.make_async_copy(emb_hbm.at[cols_ch.at[pl.ds(b * _K, _K)]],
                              gath.at[p], sem_g.at[p]).wait()

    def _scale(b, p):
        @pl.loop(0, _K // 16)
        def _grp(g):
            v16 = vals_ch[pl.ds(b * _K + g * 16, 16)]
            for j in range(16):
                idx = jnp.full((16,), j, jnp.int32)
                vb = v16.at[idx].get(mode="promise_in_bounds")
                row = g * 16 + j
                for cc in range(_D // 16):
                    sl = pl.ds(cc * 16, 16)
                    gath[p, row, sl] = gath[p, row, sl] * vb

    def _step(b, p):
        q = (p + 2) % 3
        _wait_gather(b, p)
        _scale(b, p)
        pltpu.async_copy(gath.at[p], acc.at[rows_ch.at[b]], sem_s.at[p],
                         add=True)

        @pl.when(b >= 1)
        def _drain_prev():
            pltpu.make_async_copy(gath.at[q], acc.at[rows_ch.at[b]],
                                  sem_s.at[q]).wait()

        @pl.when(b + 2 < _CH)
        def _prefetch():
            _issue_gather(b + 2, q)

    for ch in range(_NCHK):
        if ch > 0:
            pltpu.sync_copy(rows_hbm.at[w * _NCHK + ch], rows_ch)
            pltpu.sync_copy(cols_hbm.at[pl.ds(w * _EPW + ch * _CHE, _CHE)],
                            cols_ch)
            pltpu.sync_copy(vals_hbm.at[pl.ds(w * _EPW + ch * _CHE, _CHE)],
                            vals_ch)

        _issue_gather(0, 0)
        _issue_gather(1, 1)

        @pl.loop(0, _CH // 3)
        def _main(i):
            b0 = i * 3
            for u in range(3):
                _step(b0 + u, u)

        for u in range(3 * (_CH // 3), _CH):
            _step(jnp.int32(u), u % 3)

        pltpu.make_async_copy(gath.at[(_CH - 1) % 3], acc.at[rows_ch.at[0]],
                              sem_s.at[(_CH - 1) % 3]).wait()

    plsc.subcore_barrier()

    r0 = s * _RPT
    pltpu.sync_copy(acc.at[pl.ds(r0, _RPT)], out_hbm.at[c, pl.ds(r0, _RPT)])


def _sc_hop(emb, rows3, cols2, vals2):
    return pl.kernel(
        _sc_hop_body,
        out_type=jax.ShapeDtypeStruct((_NC, _NPAD, _D), jnp.float32),
        mesh=plsc.VectorSubcoreMesh(core_axis_name="c", subcore_axis_name="s"),
        scratch_types=[
            pltpu.VMEM_SHARED((_NPAD, _D), jnp.float32),
            pltpu.VMEM((_CH, _K), jnp.int32),
            pltpu.VMEM((_CHE,), jnp.int32),
            pltpu.VMEM((_CHE,), jnp.float32),
            pltpu.VMEM((3, _K, _D), jnp.float32),
            pltpu.SemaphoreType.DMA((3,)),
            pltpu.SemaphoreType.DMA((3,)),
            pltpu.SemaphoreType.DMA,
        ],
    )(emb, rows3, cols2, vals2)


def _ln_body(p_ref, g_ref, b_ref, o_ref):
    x = p_ref[0] + p_ref[1]
    mu = jnp.mean(x, axis=-1, keepdims=True)
    xc = x - mu
    var = jnp.mean(xc * xc, axis=-1, keepdims=True)
    o_ref[...] = xc * lax.rsqrt(var + _EPS) * g_ref[...] + b_ref[...]


def _tc_ln(partials, gamma, beta):
    bm = 1000
    return pl.pallas_call(
        _ln_body,
        out_shape=jax.ShapeDtypeStruct((_N_NODES, _D), jnp.float32),
        grid=(_N_NODES // bm,),
        in_specs=[
            pl.BlockSpec((_NC, bm, _D), lambda i: (0, i, 0)),
            pl.BlockSpec((1, _D), lambda i: (0, 0)),
            pl.BlockSpec((1, _D), lambda i: (0, 0)),
        ],
        out_specs=pl.BlockSpec((bm, _D), lambda i: (i, 0)),
    )(partials, gamma.reshape(1, _D), beta.reshape(1, _D))


def kernel(user_embed, item_embed, edge_index, edge_values, gamma, beta):
    all_embed = jnp.concatenate([user_embed, item_embed], axis=0)
    npad_e = _EP - _E
    pad_rows = _N_NODES + (jnp.arange(npad_e, dtype=jnp.int32)
                           % (_NPAD - _N_NODES))
    rows = jnp.concatenate(
        [edge_index[0].astype(jnp.int32), pad_rows]).reshape(
             _NW * _NCHK, _CH, _K)
    cols = jnp.concatenate(
        [edge_index[1].astype(jnp.int32),
         jnp.arange(npad_e, dtype=jnp.int32) % _N_NODES])
    edge_values = jnp.concatenate(
        [edge_values, jnp.zeros((npad_e,), jnp.float32)])
    agg = all_embed
    embs = [all_embed]
    for _ in range(3):
        partials = _sc_hop(agg, rows, cols, edge_values)
        agg = _tc_ln(partials[:, :_N_NODES], gamma, beta)
        embs.append(agg)
    embs = jnp.stack(embs, axis=1)
    return embs[:_N_USERS], embs[_N_USERS:]


# speedup vs baseline: 2.2926x; 1.0484x over previous
import jax
import jax.numpy as jnp
from jax import lax
from jax.experimental import pallas as pl
from jax.experimental.pallas import tpu as pltpu
from jax.experimental.pallas import tpu_sc as plsc

_N_USERS = 5000
_N_NODES = 10000
_D = 128
_E = 320000
_EPS = 1e-5

_NC = 2
_NS = 16
_NW = _NC * _NS
_K = 96
_NB = 105
_EPW = _K * _NB
_EP = _NW * _EPW
_NPAD = 10240
_RPT = _NPAD // _NS
_CH = 15
_NCHK = _NB // _CH
_CHE = _CH * _K
_CHP = 16


def _sc_hop_body(emb_hbm, rows_hbm, cols_hbm, vals_hbm, out_hbm,
                 acc, rows_ch, cols_ch, vals_ch, gath, sem_g, sem_s, sem_i):
    c = lax.axis_index("c")
    s = lax.axis_index("s")
    w = s * _NC + c

    def _issue_chunk_load(ck, par):
        pltpu.async_copy(rows_hbm.at[w * _NCHK + ck],
                         rows_ch.at[pl.ds(par * _CHP, _CH)], sem_i)
        pltpu.async_copy(cols_hbm.at[pl.ds(w * _EPW + ck * _CHE, _CHE)],
                         cols_ch.at[pl.ds(par * _CHE, _CHE)], sem_i)
        pltpu.async_copy(vals_hbm.at[pl.ds(w * _EPW + ck * _CHE, _CHE)],
                         vals_ch.at[pl.ds(par * _CHE, _CHE)], sem_i)

    def _wait_chunk_load(par):
        pltpu.make_async_copy(rows_hbm.at[0],
                              rows_ch.at[pl.ds(par * _CHP, _CH)],
                              sem_i).wait()
        pltpu.make_async_copy(cols_hbm.at[pl.ds(0, _CHE)],
                              cols_ch.at[pl.ds(par * _CHE, _CHE)],
                              sem_i).wait()
        pltpu.make_async_copy(vals_hbm.at[pl.ds(0, _CHE)],
                              vals_ch.at[pl.ds(par * _CHE, _CHE)],
                              sem_i).wait()

    _issue_chunk_load(0, 0)

    z16 = jnp.zeros((16,), jnp.float32)

    @pl.loop(0, _K)
    def _zero_rows(r):
        for cc in range(_D // 16):
            gath[0, r, pl.ds(cc * 16, 16)] = z16

    for k in range(_RPT // _K):
        pltpu.sync_copy(gath.at[0], acc.at[pl.ds(s * _RPT + k * _K, _K)])
    _rem = _RPT - (_RPT // _K) * _K
    if _rem:
        pltpu.sync_copy(gath.at[0, pl.ds(0, _rem)],
                        acc.at[pl.ds(s * _RPT + (_RPT // _K) * _K, _rem)])
    _wait_chunk_load(0)
    plsc.subcore_barrier()

    def _issue_gather(par, lb, p):
        off = par * _CHE + lb * _K
        return pltpu.async_copy(emb_hbm.at[cols_ch.at[pl.ds(off, _K)]],
                                gath.at[p], sem_g.at[p])

    def _wait_gather(p):
        pltpu.make_async_copy(emb_hbm.at[cols_ch.at[pl.ds(0, _K)]],
                              gath.at[p], sem_g.at[p]).wait()

    def _scale(par, l, p):
        @pl.loop(0, _K // 16)
        def _grp(g):
            v16 = vals_ch[pl.ds(par * _CHE + l * _K + g * 16, 16)]
            for j in range(16):
                idx = jnp.full((16,), j, jnp.int32)
                vb = v16.at[idx].get(mode="promise_in_bounds")
                row = g * 16 + j
                for cc in range(_D // 16):
                    sl = pl.ds(cc * 16, 16)
                    gath[p, row, sl] = gath[p, row, sl] * vb

    def _drain_scatter(p):
        pltpu.make_async_copy(gath.at[p], acc.at[rows_ch.at[0]],
                              sem_s.at[p]).wait()

    def _step(ch, par, l, u):
        q = (u + 2) % 3
        g = ch * _CH + l
        _wait_gather(u)
        _scale(par, l, u)
        pltpu.async_copy(gath.at[u], acc.at[rows_ch.at[par * _CHP + l]],
                         sem_s.at[u], add=True)

        @pl.when(g >= 1)
        def _drain_prev():
            _drain_scatter(q)

        @pl.when(l + 2 < _CH)
        def _prefetch_same():
            _issue_gather(par, l + 2, q)

        @pl.when(jnp.logical_and(l + 2 >= _CH, ch < _NCHK - 1))
        def _prefetch_next():
            _issue_gather(1 - par, l + 2 - _CH, q)

    _issue_gather(0, 0, 0)
    _issue_gather(0, 1, 1)

    @pl.loop(0, _NCHK)
    def _chunks(ch):
        par = lax.rem(ch, 2)

        @pl.loop(0, _CH // 3)
        def _main(i):
            l0 = i * 3
            _step(ch, par, l0, 0)

            @pl.when(jnp.logical_and(i == 0, ch < _NCHK - 1))
            def _load_next():
                _issue_chunk_load(ch + 1, 1 - par)

            @pl.when(jnp.logical_and(i == _CH // 3 - 1, ch < _NCHK - 1))
            def _wait_next():
                _wait_chunk_load(1 - par)

            _step(ch, par, l0 + 1, 1)
            _step(ch, par, l0 + 2, 2)

    _drain_scatter((_NB - 1) % 3)
    plsc.subcore_barrier()

    r0 = s * _RPT
    pltpu.sync_copy(acc.at[pl.ds(r0, _RPT)], out_hbm.at[c, pl.ds(r0, _RPT)])


def _sc_hop(emb, rows3, cols2, vals2):
    return pl.kernel(
        _sc_hop_body,
        out_type=jax.ShapeDtypeStruct((_NC, _NPAD, _D), jnp.float32),
        mesh=plsc.VectorSubcoreMesh(core_axis_name="c", subcore_axis_name="s"),
        scratch_types=[
            pltpu.VMEM_SHARED((_NPAD, _D), jnp.float32),
            pltpu.VMEM((2 * _CHP, _K), jnp.int32),
            pltpu.VMEM((2 * _CHE,), jnp.int32),
            pltpu.VMEM((2 * _CHE,), jnp.float32),
            pltpu.VMEM((3, _K, _D), jnp.float32),
            pltpu.SemaphoreType.DMA((3,)),
            pltpu.SemaphoreType.DMA((3,)),
            pltpu.SemaphoreType.DMA,
        ],
    )(emb, rows3, cols2, vals2)


def _ln_body(p_ref, g_ref, b_ref, o_ref):
    x = p_ref[0] + p_ref[1]
    mu = jnp.mean(x, axis=-1, keepdims=True)
    xc = x - mu
    var = jnp.mean(xc * xc, axis=-1, keepdims=True)
    o_ref[...] = xc * lax.rsqrt(var + _EPS) * g_ref[...] + b_ref[...]


def _tc_ln(partials, gamma, beta):
    bm = 1000
    return pl.pallas_call(
        _ln_body,
        out_shape=jax.ShapeDtypeStruct((_N_NODES, _D), jnp.float32),
        grid=(_N_NODES // bm,),
        in_specs=[
            pl.BlockSpec((_NC, bm, _D), lambda i: (0, i, 0)),
            pl.BlockSpec((1, _D), lambda i: (0, 0)),
            pl.BlockSpec((1, _D), lambda i: (0, 0)),
        ],
        out_specs=pl.BlockSpec((bm, _D), lambda i: (i, 0)),
    )(partials, gamma.reshape(1, _D), beta.reshape(1, _D))


def kernel(user_embed, item_embed, edge_index, edge_values, gamma, beta):
    all_embed = jnp.concatenate([user_embed, item_embed], axis=0)
    npad_e = _EP - _E
    pad_rows = _N_NODES + (jnp.arange(npad_e, dtype=jnp.int32)
                           % (_NPAD - _N_NODES))
    rows = jnp.concatenate(
        [edge_index[0].astype(jnp.int32), pad_rows]).reshape(
             _NW * _NCHK, _CH, _K)
    cols = jnp.concatenate(
        [edge_index[1].astype(jnp.int32),
         jnp.arange(npad_e, dtype=jnp.int32) % _N_NODES])
    edge_values = jnp.concatenate(
        [edge_values, jnp.zeros((npad_e,), jnp.float32)])
    agg = all_embed
    embs = [all_embed]
    for _ in range(3):
        partials = _sc_hop(agg, rows, cols, edge_values)
        agg = _tc_ln(partials[:, :_N_NODES], gamma, beta)
        embs.append(agg)
    embs = jnp.stack(embs, axis=1)
    return embs[:_N_USERS], embs[_N_USERS:]
